# baseline (speedup 1.0000x reference)
import jax
import jax.numpy as jnp
from jax import lax
from jax.experimental import pallas as pl
from jax.experimental.pallas import tpu as pltpu
from jax.experimental.pallas import tpu_sc as plsc

D = 512
DH = 128
G = 512
BLK = 512
N_NODES = 50000

NC = 2
NS = 16
NW = NC * NS
NPAD = 50176
NPART = 2
NH = NPAD // NPART
NBLKP = NH // BLK
NRG = 8
NCG = 4
CSL = D // NCG
NSL = CSL // 16
RPG = NH // NRG
CH = 224
NCHUNK = RPG // CH

_MESH = plsc.VectorSubcoreMesh(core_axis_name="c", subcore_axis_name="s")


def _make_gating_body(base_row):
    def _gating_body(idx_ref, x_ref, w1_ref, b1_ref, w2_ref, b2_ref,
                     scaled_ref, cnt_ref):
        i = pl.program_id(0)
        x = x_ref[...]
        rid2 = (lax.broadcasted_iota(jnp.int32, (BLK, 1), 0)
                + (base_row + i * BLK))
        valid2 = rid2 < N_NODES
        x = jnp.where(valid2, x, 0.0)
        h = lax.dot_general(x, w1_ref[...], (((1,), (0,)), ((), ())),
                            preferred_element_type=jnp.float32)
        h = jnp.maximum(h + b1_ref[...], 0.0)
        fc = lax.dot_general(h, w2_ref[...], (((1,), (0,)), ((), ())),
                             preferred_element_type=jnp.float32)
        fc = jnp.tanh(fc + b2_ref[...])
        scaled_ref[...] = (fc + 1.0) * x

        idx = idx_ref[0, 0, :]
        gids = lax.broadcasted_iota(jnp.int32, (BLK, G), 1)
        onehot = ((idx[:, None] == gids) & valid2).astype(jnp.float32)
        cnt_c = lax.dot_general(onehot, jnp.ones((BLK, 1), jnp.float32),
                                (((0,), (0,)), ((), ())),
                                preferred_element_type=jnp.float32)

        @pl.when(i == 0)
        def _init():
            cnt_ref[...] = jnp.zeros_like(cnt_ref)

        cnt_ref[...] += cnt_c

    return _gating_body


def _make_sc_segsum_body(use_coef):
    def body_fn(scaled_hbm, idx_hbm, coef_hbm, zeros_hbm, out_hbm,
                buf0, buf1, idxb0, idxb1, coefb0, coefb1, acc, sem0, sem1):
        cid = lax.axis_index("c")
        sid = lax.axis_index("s")
        wid = sid * NC + cid
        rg = wid // NCG
        cg = wid % NCG
        pltpu.sync_copy(zeros_hbm, acc)
        base = rg * RPG
        bufs = (buf0, buf1)
        idxbs = (idxb0, idxb1)
        coefbs = (coefb0, coefb1)
        sems = (sem0, sem1)

        def start(chunk, par):
            off = base + chunk * CH
            pltpu.async_copy(
                scaled_hbm.at[pl.ds(off, CH), pl.ds(cg * CSL, CSL)],
                bufs[par], sems[par])
            pltpu.async_copy(idx_hbm.at[pl.ds(off, CH)], idxbs[par],
                             sems[par])
            if use_coef:
                pltpu.async_copy(coef_hbm.at[pl.ds(off, CH)], coefbs[par],
                                 sems[par])

        def drain(par):
            pltpu.make_async_copy(
                scaled_hbm.at[pl.ds(0, CH), pl.ds(0, CSL)],
                bufs[par], sems[par]).wait()
            pltpu.make_async_copy(idx_hbm.at[pl.ds(0, CH)], idxbs[par],
                                  sems[par]).wait()
            if use_coef:
                pltpu.make_async_copy(coef_hbm.at[pl.ds(0, CH)], coefbs[par],
                                      sems[par]).wait()

        def consume(par, carry):
            buf = bufs[par]
            idxb = idxbs[par]
            coefb = coefbs[par]

            def grp(t, c2):
                regs = list(c2[:NSL])
                prev_g = c2[NSL]
                idx16 = idxb[pl.ds(t * 16, 16)]
                if use_coef:
                    coef16 = coefb[pl.ds(t * 16, 16)]
                for k in range(8):
                    g0 = idx16[2 * k]
                    g1 = idx16[2 * k + 1]
                    r0 = t * 16 + 2 * k
                    r1 = r0 + 1
                    neq = g0 != prev_g

                    @pl.when(neq & (prev_g >= 0))
                    def _flush():
                        for j in range(NSL):
                            sl = pl.ds(j * 16, 16)
                            acc[prev_g, sl] = acc[prev_g, sl] + regs[j]

                    keep = jnp.full((16,), jnp.where(neq, 0.0, 1.0),
                                    jnp.float32)
                    same = g0 == g1
                    msame = jnp.full((16,), jnp.where(same, 1.0, 0.0),
                                     jnp.float32)
                    t_run = []
                    for j in range(NSL):
                        sl = pl.ds(j * 16, 16)
                        if use_coef:
                            row0 = buf[r0, sl] * jnp.full((16,), coef16[2 * k],
                                                          jnp.float32)
                        else:
                            row0 = buf[r0, sl]
                        t_run.append(row0 + keep * regs[j])

                    @pl.when(jnp.logical_not(same))
                    def _flush_mid():
                        for j in range(NSL):
                            sl = pl.ds(j * 16, 16)
                            acc[g0, sl] = acc[g0, sl] + t_run[j]

                    for j in range(NSL):
                        sl = pl.ds(j * 16, 16)
                        if use_coef:
                            row1 = buf[r1, sl] * jnp.full((16,),
                                                          coef16[2 * k + 1],
                                                          jnp.float32)
                        else:
                            row1 = buf[r1, sl]
                        regs[j] = row1 + msame * t_run[j]
                    prev_g = g1
                return (*regs, prev_g)

            return lax.fori_loop(0, CH // 16, grp, carry)

        start(0, 0)
        start(1, 1)
        zero = jnp.zeros((16,), jnp.float32)
        carry0 = (*([zero] * NSL), jnp.int32(-1))

        def pair(p, carry):
            c = carry
            for par in range(2):
                chunk = p * 2 + par
                drain(par)
                c = consume(par, c)
                nxt = chunk + 2

                @pl.when(nxt < NCHUNK)
                def _prefetch():
                    start(nxt, par)
            return c

        carry = lax.fori_loop(0, NCHUNK // 2, pair, carry0)

        prev_g = carry[NSL]

        @pl.when(prev_g >= 0)
        def _final_flush():
            for j in range(NSL):
                sl = pl.ds(j * 16, 16)
                acc[prev_g, sl] = acc[prev_g, sl] + carry[j]

        pltpu.sync_copy(acc, out_hbm.at[rg, cg])

    return body_fn


_sc_segsum_plain = _make_sc_segsum_body(False)
_sc_segsum_coef = _make_sc_segsum_body(True)


def _context_body(segacc_a_ref, segacc_b_ref, cnt_a_ref, cnt_b_ref,
                  wm_ref, gc_ref):
    cnt = cnt_a_ref[...] + cnt_b_ref[...]
    gcv = jnp.zeros((G, D), jnp.float32)
    for j in range(NCG):
        seg = segacc_a_ref[0, j] + segacc_b_ref[0, j]
        for r in range(1, NRG):
            seg = seg + segacc_a_ref[r, j] + segacc_b_ref[r, j]
        mean = seg / cnt
        gcv = gcv + lax.dot_general(
            mean, wm_ref[:, j * CSL:(j + 1) * CSL], (((1,), (1,)), ((), ())),
            preferred_element_type=jnp.float32)
    gc_ref[...] = jnp.tanh(gcv)


def _coef_body(idx_ref, scaled_ref, gc_ref, coef_ref):
    idx = idx_ref[0, 0, :]
    gids = lax.broadcasted_iota(jnp.int32, (BLK, G), 1)
    onehot = (idx[:, None] == gids).astype(jnp.float32)
    dots = lax.dot_general(scaled_ref[...], gc_ref[...],
                           (((1,), (1,)), ((), ())),
                           preferred_element_type=jnp.float32)
    s = jnp.sum(dots * onehot, axis=1)
    coef_ref[0, 0, :] = 1.0 / (1.0 + jnp.exp(-s))


def _combine_body(acc_a_ref, acc_b_ref, out_ref):
    for j in range(NCG):
        s = acc_a_ref[0, j] + acc_b_ref[0, j]
        for r in range(1, NRG):
            s = s + acc_a_ref[r, j] + acc_b_ref[r, j]
        out_ref[:, j * CSL:(j + 1) * CSL] = s


def kernel(node_features, graph_idx, num_graphs, W_g1, b_g1, W_g2, b_g2, W_mean):
    x = node_features.astype(jnp.float32)
    idx = jnp.pad(graph_idx.astype(jnp.int32), (0, NPAD - x.shape[0]))
    idx3 = idx.reshape(NPAD // BLK, 1, BLK)
    w1t = W_g1.T
    w2t = W_g2.T
    b1 = b_g1.reshape(1, DH)
    b2 = b_g2.reshape(1, D)
    zeros_acc = jnp.zeros((G, CSL), jnp.float32)
    ones_n = jnp.ones((NH,), jnp.float32)
    idx_p = [idx[p * NH:(p + 1) * NH] for p in range(NPART)]

    sc_scratch = [
        pltpu.VMEM((CH, CSL), jnp.float32),
        pltpu.VMEM((CH, CSL), jnp.float32),
        pltpu.VMEM((CH,), jnp.int32),
        pltpu.VMEM((CH,), jnp.int32),
        pltpu.VMEM((CH,), jnp.float32),
        pltpu.VMEM((CH,), jnp.float32),
        pltpu.VMEM((G, CSL), jnp.float32),
        pltpu.SemaphoreType.DMA,
        pltpu.SemaphoreType.DMA,
    ]
    acc_type = jax.ShapeDtypeStruct((NRG, NCG, G, CSL), jnp.float32)

    scaled_p, cnt_p = [], []
    for p in range(NPART):
        scaled, cnt = pl.pallas_call(
            _make_gating_body(p * NH),
            grid=(NBLKP,),
            in_specs=[
                pl.BlockSpec((1, 1, BLK), lambda i, p=p: (i + p * NBLKP, 0, 0)),
                pl.BlockSpec((BLK, D), lambda i, p=p: (i + p * NBLKP, 0)),
                pl.BlockSpec((D, DH), lambda i: (0, 0)),
                pl.BlockSpec((1, DH), lambda i: (0, 0)),
                pl.BlockSpec((DH, D), lambda i: (0, 0)),
                pl.BlockSpec((1, D), lambda i: (0, 0)),
            ],
            out_specs=[
                pl.BlockSpec((BLK, D), lambda i: (i, 0)),
                pl.BlockSpec((G, 1), lambda i: (0, 0)),
            ],
            out_shape=[
                jax.ShapeDtypeStruct((NH, D), jnp.float32),
                jax.ShapeDtypeStruct((G, 1), jnp.float32),
            ],
            compiler_params=pltpu.CompilerParams(
                dimension_semantics=("arbitrary",)),
        )(idx3, x, w1t, b1, w2t, b2)
        scaled_p.append(scaled)
        cnt_p.append(cnt)

    segacc_p = [
        pl.kernel(_sc_segsum_plain, out_type=acc_type, mesh=_MESH,
                  scratch_types=sc_scratch)(
            scaled_p[p], idx_p[p], ones_n, zeros_acc)
        for p in range(NPART)
    ]

    gc = pl.pallas_call(
        _context_body,
        out_shape=jax.ShapeDtypeStruct((G, D), jnp.float32),
    )(segacc_p[0], segacc_p[1], cnt_p[0], cnt_p[1], W_mean)

    coef_p = []
    for p in range(NPART):
        coef3 = pl.pallas_call(
            _coef_body,
            grid=(NBLKP,),
            in_specs=[
                pl.BlockSpec((1, 1, BLK), lambda i, p=p: (i + p * NBLKP, 0, 0)),
                pl.BlockSpec((BLK, D), lambda i: (i, 0)),
                pl.BlockSpec((G, D), lambda i: (0, 0)),
            ],
            out_specs=pl.BlockSpec((1, 1, BLK), lambda i: (i, 0, 0)),
            out_shape=jax.ShapeDtypeStruct((NBLKP, 1, BLK), jnp.float32),
            compiler_params=pltpu.CompilerParams(
                dimension_semantics=("arbitrary",)),
        )(idx3, scaled_p[p], gc)
        coef_p.append(coef3.reshape(NH))

    outacc_p = [
        pl.kernel(_sc_segsum_coef, out_type=acc_type, mesh=_MESH,
                  scratch_types=sc_scratch)(
            scaled_p[p], idx_p[p], coef_p[p], zeros_acc)
        for p in range(NPART)
    ]

    out = pl.pallas_call(
        _combine_body,
        out_shape=jax.ShapeDtypeStruct((G, D), jnp.float32),
    )(outacc_p[0], outacc_p[1])

    return out
